# baseline (speedup 1.0000x reference)
import jax
import jax.numpy as jnp
from jax.experimental import pallas as pl

_TOKEN_BLOCK = 4096


def _probe_body(x_ref, o_ref):
    x = x_ref[...]
    acc = x
    for _ in range(8):
        acc = acc * 1.0000001 + 0.5
    o_ref[...] = jnp.full((8, 128), jnp.sum(acc), jnp.float32)


def kernel(x, weight, bias):
    flat = x.reshape(-1, x.shape[-1])
    T, H = flat.shape
    tb = _TOKEN_BLOCK
    nb = T // tb
    out = pl.pallas_call(
        _probe_body,
        grid=(nb,),
        in_specs=[pl.BlockSpec((tb, H), lambda i: (i, 0))],
        out_specs=pl.BlockSpec((8, 128), lambda i: (i, 0)),
        out_shape=jax.ShapeDtypeStruct((nb * 8, 128), jnp.float32),
    )(flat)
    return out


# speedup vs baseline: 2.8166x; 2.8166x over previous
import jax
import jax.numpy as jnp
from jax.experimental import pallas as pl

_NUM_EXPERTS = 8
_TOKEN_BLOCK = 4096


def _router_body(x_ref, w_ref, b_ref, full_t_ref, idx_t_ref):
    x = x_ref[...]
    w = w_ref[...]
    logits = jax.lax.dot_general(
        x, w, (((1,), (1,)), ((), ())),
        preferred_element_type=jnp.float32,
    )
    lt = logits.T + b_ref[...]
    m = jnp.max(lt, axis=0, keepdims=True)
    denom = jnp.sum(jnp.exp(lt - m), axis=0, keepdims=True)
    top_score = 1.0 / denom
    subl = jax.lax.broadcasted_iota(jnp.int32, lt.shape, 0)
    idx = jnp.min(jnp.where(lt == m, subl, _NUM_EXPERTS),
                  axis=0, keepdims=True)
    full_t_ref[...] = jnp.where(subl == idx, top_score, 0.0)
    idx_t_ref[...] = idx


def kernel(x, weight, bias):
    flat = x.reshape(-1, x.shape[-1])
    T, H = flat.shape
    E = weight.shape[0]
    b = bias.reshape(E, 1)
    tb = _TOKEN_BLOCK
    full_t, idx_t = pl.pallas_call(
        _router_body,
        grid=(T // tb,),
        in_specs=[
            pl.BlockSpec((tb, H), lambda i: (i, 0)),
            pl.BlockSpec((E, H), lambda i: (0, 0)),
            pl.BlockSpec((E, 1), lambda i: (0, 0)),
        ],
        out_specs=[
            pl.BlockSpec((E, tb), lambda i: (0, i)),
            pl.BlockSpec((1, tb), lambda i: (0, i)),
        ],
        out_shape=[
            jax.ShapeDtypeStruct((E, T), jnp.float32),
            jax.ShapeDtypeStruct((1, T), jnp.int32),
        ],
    )(flat, weight, b)
    return (full_t.T, idx_t.reshape(T, 1))
